# baseline (speedup 1.0000x reference)
import functools

import jax
import jax.numpy as jnp
from jax import lax
from jax.experimental import pallas as pl
from jax.experimental.pallas import tpu as pltpu
from jax.experimental.pallas import tpu_sc as plsc

NC = 2
NS = 16
NW = NC * NS
CH = 128
BV = 32768
SUB_SHIFT = (BV // 4).bit_length() - 1
PAR_SHIFT = (BV // 2).bit_length() - 1


def _reformat_body(xt_ref, out_ref):
    x = xt_ref[...]
    z = jnp.concatenate([x[:, :BV // 2], x[:, BV // 2:]], axis=0)
    zt = z.T
    zi = lax.bitcast_convert_type(zt, jnp.int32)
    zb = lax.shift_right_logical(
        zi + 0x7FFF + (lax.shift_right_logical(zi, 16) & 1), 16)
    out_ref[...] = zb[:BV // 4] | (zb[BV // 4:] << 16)


def _tc_reformat(table_t):
    emb, vocab = table_t.shape
    grid = pl.cdiv(vocab, BV)
    return pl.pallas_call(
        _reformat_body,
        grid=(grid,),
        in_specs=[pl.BlockSpec((emb, BV), lambda i: (0, i))],
        out_specs=pl.BlockSpec((BV // 4, 2 * emb), lambda i: (i, 0)),
        out_shape=jax.ShapeDtypeStruct((grid * BV // 4, 2 * emb), jnp.int32),
    )(table_t)


def _gather_body(idx_hbm, table_hbm, out_hbm, idx_v, buf0, buf1, sem0, sem1,
                 *, n_chunk):
    wid = lax.axis_index("s") * NC + lax.axis_index("c")
    rows_w = n_chunk * CH
    base = wid * rows_w
    pltpu.sync_copy(idx_hbm.at[pl.ds(base, rows_w)], idx_v)

    bufs = (buf0, buf1)
    sems = (sem0, sem1)

    def start(j):
        return pltpu.async_copy(
            table_hbm.at[idx_v.at[pl.ds(j * CH, CH)]], bufs[j % 2], sems[j % 2])

    descs = [None] * n_chunk
    descs[0] = start(0)
    for j in range(n_chunk):
        if j + 1 < n_chunk:
            descs[j + 1] = start(j + 1)
        descs[j].wait()
        pltpu.sync_copy(bufs[j % 2], out_hbm.at[pl.ds(base + j * CH, CH)])


def _sc_gather(idx, table2, n_rows):
    n_chunk = n_rows // (NW * CH)
    mesh = plsc.VectorSubcoreMesh(core_axis_name="c", subcore_axis_name="s")
    body = functools.partial(_gather_body, n_chunk=n_chunk)
    return pl.kernel(
        body,
        out_type=jax.ShapeDtypeStruct((n_rows, 128), jnp.int32),
        mesh=mesh,
        scratch_types=[
            pltpu.VMEM((n_rows // NW,), jnp.int32),
            pltpu.VMEM((CH, 128), jnp.int32),
            pltpu.VMEM((CH, 128), jnp.int32),
            pltpu.SemaphoreType.DMA,
            pltpu.SemaphoreType.DMA,
        ],
    )(idx, table2)


def _mlp_body(x3_ref, toks_ref, w13_ref, b1_ref, w2_ref, b2_ref,
              out_ref, *, seq, hidden):
    blk = out_ref.shape[0]
    acc = jnp.zeros((blk, hidden), jnp.float32) + b1_ref[...]
    for p in range(seq):
        xp = x3_ref[p]
        tok = toks_ref[:, p].reshape(blk, 1)
        subp = lax.shift_right_logical(tok, SUB_SHIFT) & 1
        bits = lax.shift_left(xp, (1 - subp) * 16) & jnp.int32(-65536)
        xf = lax.bitcast_convert_type(bits, jnp.float32)
        parp = lax.shift_right_logical(tok, PAR_SHIFT) & 1
        sel = jnp.where(parp > 0, xf[:, 64:], xf[:, :64])
        acc += jnp.dot(sel.astype(jnp.bfloat16), w13_ref[p],
                       preferred_element_type=jnp.float32)
    h = jnp.maximum(acc, 0.0)
    logits = jnp.dot(h, w2_ref[...],
                     preferred_element_type=jnp.float32) + b2_ref[...]
    m = jnp.max(logits, axis=1, keepdims=True)
    e = logits - m
    lse = jnp.log(jnp.sum(jnp.exp(e), axis=1, keepdims=True))
    out_ref[...] = e - lse


def _tc_mlp(x3, toks, w1, b1, w2, b2, num_tags):
    seq, bs, _ = x3.shape
    in_dim, hidden = w1.shape
    emb = in_dim // seq
    blk = 2048
    grid = bs // blk
    body = functools.partial(_mlp_body, seq=seq, hidden=hidden)
    return pl.pallas_call(
        body,
        grid=(grid,),
        in_specs=[
            pl.BlockSpec((seq, blk, 128), lambda i: (0, i, 0)),
            pl.BlockSpec((blk, seq), lambda i: (i, 0)),
            pl.BlockSpec((seq, emb, hidden), lambda i: (0, 0, 0)),
            pl.BlockSpec((1, hidden), lambda i: (0, 0)),
            pl.BlockSpec((hidden, num_tags), lambda i: (0, 0)),
            pl.BlockSpec((1, num_tags), lambda i: (0, 0)),
        ],
        out_specs=pl.BlockSpec((blk, num_tags), lambda i: (i, 0)),
        out_shape=jax.ShapeDtypeStruct((bs, num_tags), jnp.float32),
    )(x3, toks, w1.reshape(seq, emb, hidden).astype(jnp.bfloat16),
      b1.reshape(1, hidden), w2, b2.reshape(1, num_tags))


def kernel(Xtoks_IDs, emb_table, W1, b1, W2, b2):
    bs, seq = Xtoks_IDs.shape
    vocab, emb = emb_table.shape
    num_tags = W2.shape[1]

    toks_t = Xtoks_IDs.astype(jnp.int32).T
    blk_i = toks_t // BV
    l2 = (toks_t % BV) % (BV // 2)
    word_idx = (blk_i * (BV // 4) + l2 % (BV // 4)).reshape(-1)

    table2 = _tc_reformat(emb_table.T)

    toks32 = Xtoks_IDs.astype(jnp.int32)
    word2 = word_idx.reshape(seq, bs)
    hb = bs // 4
    outs = []
    for k in range(4):
        wk = word2[:, k * hb:(k + 1) * hb].reshape(-1)
        rows = _sc_gather(wk, table2, seq * hb)
        x3 = rows.reshape(seq, hb, 2 * emb)
        outs.append(_tc_mlp(x3, toks32[k * hb:(k + 1) * hb], W1, b1, W2, b2,
                            num_tags))
    return jnp.concatenate(outs, axis=0)


# speedup vs baseline: 1.0628x; 1.0628x over previous
import functools

import jax
import jax.numpy as jnp
from jax import lax
from jax.experimental import pallas as pl
from jax.experimental.pallas import tpu as pltpu
from jax.experimental.pallas import tpu_sc as plsc

NC = 2
NS = 16
NW = NC * NS
CH = 128
BV = 32768
SUB_SHIFT = (BV // 4).bit_length() - 1
PAR_SHIFT = (BV // 2).bit_length() - 1


def _reformat_body(xt_ref, out_ref):
    x = xt_ref[...]
    z = jnp.concatenate([x[:, :BV // 2], x[:, BV // 2:]], axis=0)
    zt = z.T
    zi = lax.bitcast_convert_type(zt, jnp.int32)
    zb = lax.shift_right_logical(
        zi + 0x7FFF + (lax.shift_right_logical(zi, 16) & 1), 16)
    out_ref[...] = zb[:BV // 4] | (zb[BV // 4:] << 16)


def _tc_reformat(table_t):
    emb, vocab = table_t.shape
    grid = pl.cdiv(vocab, BV)
    return pl.pallas_call(
        _reformat_body,
        grid=(grid,),
        in_specs=[pl.BlockSpec((emb, BV), lambda i: (0, i))],
        out_specs=pl.BlockSpec((BV // 4, 2 * emb), lambda i: (i, 0)),
        out_shape=jax.ShapeDtypeStruct((grid * BV // 4, 2 * emb), jnp.int32),
    )(table_t)


def _gather_body(idx_hbm, table_hbm, out_hbm, idx_v,
                 buf0, buf1, buf2, buf3, sem0, sem1, sem2, sem3,
                 osem0, osem1, osem2, osem3, *, n_chunk):
    wid = lax.axis_index("s") * NC + lax.axis_index("c")
    rows_w = n_chunk * CH
    base = wid * rows_w
    pltpu.sync_copy(idx_hbm.at[pl.ds(base, rows_w)], idx_v)

    bufs = (buf0, buf1, buf2, buf3)
    sems = (sem0, sem1, sem2, sem3)
    osems = (osem0, osem1, osem2, osem3)

    def start_in(j):
        return pltpu.async_copy(
            table_hbm.at[idx_v.at[pl.ds(j * CH, CH)]], bufs[j % 4], sems[j % 4])

    def start_out(j):
        return pltpu.async_copy(
            bufs[j % 4], out_hbm.at[pl.ds(base + j * CH, CH)], osems[j % 4])

    descs_in = [None] * n_chunk
    descs_out = [None] * n_chunk
    for j in range(min(2, n_chunk)):
        descs_in[j] = start_in(j)
    for j in range(n_chunk):
        if j + 2 < n_chunk:
            if j >= 2:
                descs_out[j - 2].wait()
            descs_in[j + 2] = start_in(j + 2)
        descs_in[j].wait()
        descs_out[j] = start_out(j)
    for j in range(max(0, n_chunk - 2), n_chunk):
        descs_out[j].wait()


def _sc_gather(idx, table2, n_rows):
    n_chunk = n_rows // (NW * CH)
    mesh = plsc.VectorSubcoreMesh(core_axis_name="c", subcore_axis_name="s")
    body = functools.partial(_gather_body, n_chunk=n_chunk)
    return pl.kernel(
        body,
        out_type=jax.ShapeDtypeStruct((n_rows, 128), jnp.int32),
        mesh=mesh,
        scratch_types=[
            pltpu.VMEM((n_rows // NW,), jnp.int32),
            pltpu.VMEM((CH, 128), jnp.int32),
            pltpu.VMEM((CH, 128), jnp.int32),
            pltpu.VMEM((CH, 128), jnp.int32),
            pltpu.VMEM((CH, 128), jnp.int32),
            pltpu.SemaphoreType.DMA,
            pltpu.SemaphoreType.DMA,
            pltpu.SemaphoreType.DMA,
            pltpu.SemaphoreType.DMA,
            pltpu.SemaphoreType.DMA,
            pltpu.SemaphoreType.DMA,
            pltpu.SemaphoreType.DMA,
            pltpu.SemaphoreType.DMA,
        ],
    )(idx, table2)


def _mlp_body(x3_ref, toks_ref, w13_ref, b1_ref, w2_ref, b2_ref,
              out_ref, *, seq, hidden):
    blk = out_ref.shape[0]
    acc = jnp.zeros((blk, hidden), jnp.float32) + b1_ref[...]
    for p in range(seq):
        xp = x3_ref[p]
        tok = toks_ref[:, p].reshape(blk, 1)
        subp = lax.shift_right_logical(tok, SUB_SHIFT) & 1
        bits = lax.shift_left(xp, (1 - subp) * 16) & jnp.int32(-65536)
        xf = lax.bitcast_convert_type(bits, jnp.float32)
        parp = lax.shift_right_logical(tok, PAR_SHIFT) & 1
        sel = jnp.where(parp > 0, xf[:, 64:], xf[:, :64])
        acc += jnp.dot(sel.astype(jnp.bfloat16), w13_ref[p],
                       preferred_element_type=jnp.float32)
    h = jnp.maximum(acc, 0.0)
    logits = jnp.dot(h, w2_ref[...],
                     preferred_element_type=jnp.float32) + b2_ref[...]
    m = jnp.max(logits, axis=1, keepdims=True)
    e = logits - m
    lse = jnp.log(jnp.sum(jnp.exp(e), axis=1, keepdims=True))
    out_ref[...] = e - lse


def _tc_mlp(x3, toks, w1, b1, w2, b2, num_tags):
    seq, bs, _ = x3.shape
    in_dim, hidden = w1.shape
    emb = in_dim // seq
    blk = 2048
    grid = bs // blk
    body = functools.partial(_mlp_body, seq=seq, hidden=hidden)
    return pl.pallas_call(
        body,
        grid=(grid,),
        in_specs=[
            pl.BlockSpec((seq, blk, 128), lambda i: (0, i, 0)),
            pl.BlockSpec((blk, seq), lambda i: (i, 0)),
            pl.BlockSpec((seq, emb, hidden), lambda i: (0, 0, 0)),
            pl.BlockSpec((1, hidden), lambda i: (0, 0)),
            pl.BlockSpec((hidden, num_tags), lambda i: (0, 0)),
            pl.BlockSpec((1, num_tags), lambda i: (0, 0)),
        ],
        out_specs=pl.BlockSpec((blk, num_tags), lambda i: (i, 0)),
        out_shape=jax.ShapeDtypeStruct((bs, num_tags), jnp.float32),
    )(x3, toks, w1.reshape(seq, emb, hidden).astype(jnp.bfloat16),
      b1.reshape(1, hidden), w2, b2.reshape(1, num_tags))


def kernel(Xtoks_IDs, emb_table, W1, b1, W2, b2):
    bs, seq = Xtoks_IDs.shape
    vocab, emb = emb_table.shape
    num_tags = W2.shape[1]

    toks_t = Xtoks_IDs.astype(jnp.int32).T
    blk_i = toks_t // BV
    l2 = (toks_t % BV) % (BV // 2)
    word_idx = (blk_i * (BV // 4) + l2 % (BV // 4)).reshape(-1)

    table2 = _tc_reformat(emb_table.T)

    toks32 = Xtoks_IDs.astype(jnp.int32)
    word2 = word_idx.reshape(seq, bs)
    hb = bs // 2
    outs = []
    for k in range(2):
        wk = word2[:, k * hb:(k + 1) * hb].reshape(-1)
        rows = _sc_gather(wk, table2, seq * hb)
        x3 = rows.reshape(seq, hb, 2 * emb)
        outs.append(_tc_mlp(x3, toks32[k * hb:(k + 1) * hb], W1, b1, W2, b2,
                            num_tags))
    return jnp.concatenate(outs, axis=0)
